# Initial kernel scaffold; baseline (speedup 1.0000x reference)
import functools

import jax
import jax.numpy as jnp
from jax import lax
from jax.experimental import pallas as pl
from jax.experimental.pallas import tpu as pltpu
from jax.experimental.pallas import tpu_sc as plsc

N = 100000
NSP = 100
TBL = 101
L = 16
NC, NS = 2, 16
NW = NC * NS
C = 400
NCHUNK = N // C
GROUPS = C // L
MAX_ITERS = (NCHUNK + NW - 1) // NW


def _body(species_hbm, s2i_hbm, out_hbm,
          s2i_v, sp_v, pos0, pos1, buf0, buf1, sem0, sem1):
  wid = lax.axis_index("s") * NC + lax.axis_index("c")
  my_cnt = (NCHUNK - wid + NW - 1) // NW

  bufs = (buf0, buf1)
  poss = (pos0, pos1)
  sems = (sem0, sem1)

  pltpu.sync_copy(s2i_hbm, s2i_v.at[pl.ds(0, TBL)])

  zeros = jnp.zeros((L,), jnp.int32)
  ones = jnp.ones((L,), jnp.int32)
  lane = lax.iota(jnp.int32, L)

  def _zero(j, _):
    buf0[pl.ds(j * L, L)] = zeros
    buf1[pl.ds(j * L, L)] = zeros
    return 0
  lax.fori_loop(0, (C * NSP) // L, _zero, 0)

  for i in range(MAX_ITERS):
    b = i % 2
    buf, pos, sem = bufs[b], poss[b], sems[b]

    @pl.when(i < my_cnt)
    def _():
      k = wid + i * NW

      if i >= 2:
        pltpu.make_async_copy(buf, out_hbm.at[pl.ds(0, C * NSP)], sem).wait()

        def _restore(g, _):
          p = pos[pl.ds(g * L, L)]
          plsc.store_scatter(buf, [p], zeros)
          return 0
        lax.fori_loop(0, GROUPS, _restore, 0)

      pltpu.sync_copy(species_hbm.at[pl.ds(k * C, C)], sp_v)

      def _compute(g, _):
        sp = sp_v[pl.ds(g * L, L)]
        idx = plsc.load_gather(s2i_v, [sp])
        p = (g * L + lane) * NSP + idx
        pos[pl.ds(g * L, L)] = p
        plsc.store_scatter(buf, [p], ones)
        return 0
      lax.fori_loop(0, GROUPS, _compute, 0)

      pltpu.async_copy(buf, out_hbm.at[pl.ds(k * C * NSP, C * NSP)], sem)

  for i in range(MAX_ITERS):
    b = i % 2

    @pl.when((i < my_cnt) & (i >= my_cnt - 2))
    def _():
      pltpu.make_async_copy(
          bufs[b], out_hbm.at[pl.ds(0, C * NSP)], sems[b]).wait()


@jax.jit
def kernel(species, species_to_index):
  mesh = plsc.VectorSubcoreMesh(core_axis_name="c", subcore_axis_name="s")
  run = functools.partial(
      pl.kernel,
      out_type=jax.ShapeDtypeStruct((N * NSP,), jnp.int32),
      mesh=mesh,
      scratch_types=[
          pltpu.VMEM((TBL + 3,), jnp.int32),
          pltpu.VMEM((C,), jnp.int32),
          pltpu.VMEM((C,), jnp.int32),
          pltpu.VMEM((C,), jnp.int32),
          pltpu.VMEM((C * NSP,), jnp.int32),
          pltpu.VMEM((C * NSP,), jnp.int32),
          pltpu.SemaphoreType.DMA,
          pltpu.SemaphoreType.DMA,
      ],
  )(_body)
  out = run(species, species_to_index)
  return out.reshape(N, NSP)


# speedup vs baseline: 2.9913x; 2.9913x over previous
import functools

import jax
import jax.numpy as jnp
from jax import lax
from jax.experimental import pallas as pl
from jax.experimental.pallas import tpu as pltpu
from jax.experimental.pallas import tpu_sc as plsc

N = 100000
NSP = 100
TBL = 101
L = 16
NC, NS = 2, 16
NW = NC * NS
C = 400
NCHUNK = N // C
GROUPS = C // L
MAX_ITERS = (NCHUNK + NW - 1) // NW


def _body(species_hbm, s2i_hbm, out_hbm,
          s2i_v, sp_v, pos0, pos1, buf0, buf1, sem0, sem1):
  wid = lax.axis_index("s") * NC + lax.axis_index("c")

  bufs = (buf0, buf1)
  poss = (pos0, pos1)
  sems = (sem0, sem1)

  pltpu.sync_copy(s2i_hbm, s2i_v.at[pl.ds(0, TBL)])

  zeros = jnp.zeros((L,), jnp.int32)
  ones = jnp.ones((L,), jnp.int32)
  lane = lax.iota(jnp.int32, L)

  def _zero(j, _):
    buf0[pl.ds(j * L, L)] = zeros
    buf1[pl.ds(j * L, L)] = zeros
    return 0
  lax.fori_loop(0, (C * NSP) // L, _zero, 0)

  for i in range(MAX_ITERS):
    b = i % 2
    buf, pos, sem = bufs[b], poss[b], sems[b]
    k = jnp.minimum(wid + i * NW, NCHUNK - 1)

    if i >= 2:
      pltpu.make_async_copy(buf, out_hbm.at[pl.ds(0, C * NSP)], sem).wait()

      def _restore(g, _):
        p = pos[pl.ds(g * L, L)]
        plsc.store_scatter(buf, [p], zeros)
        return 0
      lax.fori_loop(0, GROUPS, _restore, 0)

    pltpu.sync_copy(species_hbm.at[pl.ds(k * C, C)], sp_v)

    def _compute(g, _):
      sp = sp_v[pl.ds(g * L, L)]
      idx = plsc.load_gather(s2i_v, [sp])
      p = (g * L + lane) * NSP + idx
      pos[pl.ds(g * L, L)] = p
      plsc.store_scatter(buf, [p], ones)
      return 0
    lax.fori_loop(0, GROUPS, _compute, 0)

    pltpu.async_copy(buf, out_hbm.at[pl.ds(k * C * NSP, C * NSP)], sem)

  for i in range(MAX_ITERS - 2, MAX_ITERS):
    pltpu.make_async_copy(
        bufs[i % 2], out_hbm.at[pl.ds(0, C * NSP)], sems[i % 2]).wait()


@jax.jit
def kernel(species, species_to_index):
  mesh = plsc.VectorSubcoreMesh(core_axis_name="c", subcore_axis_name="s")
  run = functools.partial(
      pl.kernel,
      out_type=jax.ShapeDtypeStruct((N * NSP,), jnp.int32),
      mesh=mesh,
      compiler_params=pltpu.CompilerParams(needs_layout_passes=False),
      scratch_types=[
          pltpu.VMEM((TBL + 3,), jnp.int32),
          pltpu.VMEM((C,), jnp.int32),
          pltpu.VMEM((C,), jnp.int32),
          pltpu.VMEM((C,), jnp.int32),
          pltpu.VMEM((C * NSP,), jnp.int32),
          pltpu.VMEM((C * NSP,), jnp.int32),
          pltpu.SemaphoreType.DMA,
          pltpu.SemaphoreType.DMA,
      ],
  )(_body)
  out = run(species, species_to_index)
  return out.reshape(N, NSP)
